# Initial kernel scaffold; baseline (speedup 1.0000x reference)
import jax
import jax.numpy as jnp
from jax.experimental import pallas as pl


def kernel(rand_x, rand_edge, edge_index, node_feat, edge_attr, atom_emb, bond_emb, W1, b1, g1, be1, W2, b2, go, bo, eps):
    raise NotImplementedError("write your pallas kernel here")


# speedup vs baseline: 1.9413x; 1.9413x over previous
import functools

import jax
import jax.numpy as jnp
from jax import lax
from jax.experimental import pallas as pl
from jax.experimental.pallas import tpu as pltpu
from jax.experimental.pallas import tpu_sc as plsc

N = 10000
E = 320000
H = 128
RV = 10
SH = H - RV
L = 3

NC = 2
NS = 16
NW = NC * NS
CH = 128
NCHUNK = 79
EW = NCHUNK * CH
EP = NW * EW
NP2 = 10112
RPT = NP2 // NS
CW = SH // 2
RW = RV // 2

_sc_mesh = plsc.VectorSubcoreMesh(core_axis_name="c", subcore_axis_name="s")


def _unpack2(w):
    lo = plsc.bitcast(jnp.left_shift(w, 16), jnp.float32)
    hi = plsc.bitcast(jnp.bitwise_and(w, -65536), jnp.float32)
    return lo, hi


def _msg_body(h, srcp, dstp, cidp, randp, comboP, zrows, out,
              shared, combo_v, ibuf, hbuf, rbuf, sem):
    c = lax.axis_index("c")
    s = lax.axis_index("s")
    pltpu.sync_copy(comboP, combo_v)
    pltpu.sync_copy(zrows, shared.at[pl.ds(s * RPT, RPT)])
    plsc.subcore_barrier()

    base0 = (c * NS + s) * EW

    def chunk_body(i, carry):
        base = pl.multiple_of(base0 + i * CH, CH)
        pltpu.sync_copy(srcp.at[pl.ds(base, CH)], ibuf.at[0])
        pltpu.sync_copy(dstp.at[pl.ds(base, CH)], ibuf.at[1])
        pltpu.sync_copy(cidp.at[pl.ds(base, CH)], ibuf.at[2])
        pltpu.sync_copy(randp.at[:, pl.ds(base, CH)], rbuf)
        pltpu.async_copy(h.at[ibuf.at[0]], hbuf, sem).wait()

        def group_body(g, carry2):
            gbase = pl.multiple_of(g * 16, 16)
            cvec = ibuf[2, pl.ds(gbase, 16)]
            rows = lax.iota(jnp.int32, 16) + gbase
            zero16 = jnp.zeros((16,), jnp.int32)

            def pair(d0, c0, c1):
                dsp0 = zero16 + d0
                dsp1 = dsp0 + 1
                m0 = jnp.maximum(plsc.load_gather(hbuf, [rows, dsp0]) + c0, 0.0)
                m1 = jnp.maximum(plsc.load_gather(hbuf, [rows, dsp1]) + c1, 0.0)
                plsc.store_scatter(hbuf, [rows, dsp0], m0)
                plsc.store_scatter(hbuf, [rows, dsp1], m1)

            def word_body(w, carry3):
                cw = plsc.load_gather(combo_v, [w * 512 + cvec])
                clo, chi = _unpack2(cw)
                pair(2 * w, clo, chi)
                return carry3

            lax.fori_loop(0, CW, word_body, 0)
            for j in range(RW):
                rw = plsc.load_gather(rbuf, [jnp.full((16,), j, jnp.int32), rows])
                rlo, rhi = _unpack2(rw)
                pair(SH + 2 * j, rlo, rhi)
            return carry2

        lax.fori_loop(0, CH // 16, group_body, 0)
        pltpu.sync_copy(hbuf, shared.at[ibuf.at[1]], add=True)
        return carry

    lax.fori_loop(0, NCHUNK, chunk_body, 0)
    plsc.subcore_barrier()

    nfull = RPT // CH
    for k in range(nfull):
        rb = s * RPT + k * CH
        pltpu.sync_copy(shared.at[pl.ds(rb, CH)], hbuf)
        pltpu.sync_copy(hbuf, out.at[c, pl.ds(rb, CH)])
    rem = RPT - nfull * CH
    if rem:
        rb = s * RPT + nfull * CH
        pltpu.sync_copy(shared.at[pl.ds(rb, rem)], hbuf.at[pl.ds(0, rem)])
        pltpu.sync_copy(hbuf.at[pl.ds(0, rem)], out.at[c, pl.ds(rb, rem)])


_msg_kernel = functools.partial(
    pl.kernel,
    mesh=_sc_mesh,
    compiler_params=pltpu.CompilerParams(needs_layout_passes=False),
    out_type=jax.ShapeDtypeStruct((NC, NP2, H), jnp.float32),
    scratch_types=[
        pltpu.VMEM_SHARED((NP2, H), jnp.float32),
        pltpu.VMEM((CW * 512,), jnp.int32),
        pltpu.VMEM((3, CH), jnp.int32),
        pltpu.VMEM((CH, H), jnp.float32),
        pltpu.VMEM((RW, CH), jnp.int32),
        pltpu.SemaphoreType.DMA,
    ],
)(_msg_body)


def _enc_body(nf_ref, emb_ref, rx_ref, out_ref):
    acc = rx_ref[...]
    nf = nf_ref[...]
    for i in range(9):
        col = nf[:, i][:, None]
        oh = (col == lax.broadcasted_iota(jnp.int32, (1, 64), 1)).astype(jnp.float32)
        acc = acc + jnp.dot(oh, emb_ref[i], preferred_element_type=jnp.float32)
    out_ref[...] = acc


def _mlp_body(h_ref, a0_ref, a1_ref, w1_ref, b1_ref, g1_ref, be1_ref,
              w2_ref, b2_ref, go_ref, bo_ref, eps_ref, out_ref, *, relu_out):
    x = h_ref[...]
    t = (1.0 + eps_ref[0, 0]) * x + a0_ref[...] + a1_ref[...]
    u = jnp.dot(t, w1_ref[...], preferred_element_type=jnp.float32) + b1_ref[...]
    u = jnp.maximum(g1_ref[...] * u + be1_ref[...], 0.0)
    v = jnp.dot(u, w2_ref[...], preferred_element_type=jnp.float32) + b2_ref[...]
    v = go_ref[...] * v + bo_ref[...]
    if relu_out:
        v = jnp.maximum(v, 0.0)
    out_ref[...] = v


_R = 400
_G = N // _R

_row_spec = pl.BlockSpec((_R, H), lambda i: (i, 0))
_vec_spec = pl.BlockSpec((1, H), lambda i: (0, 0))
_mat_spec = pl.BlockSpec((H, H), lambda i: (0, 0))


def _make_mlp(relu_out):
    return pl.pallas_call(
        functools.partial(_mlp_body, relu_out=relu_out),
        grid=(_G,),
        in_specs=[_row_spec, _row_spec, _row_spec, _mat_spec, _vec_spec,
                  _vec_spec, _vec_spec, _mat_spec, _vec_spec, _vec_spec,
                  _vec_spec, pl.BlockSpec((1, 1), lambda i: (0, 0))],
        out_specs=_row_spec,
        out_shape=jax.ShapeDtypeStruct((N, H), jnp.float32),
    )


_mlp_relu = _make_mlp(True)
_mlp_last = _make_mlp(False)

_encoder = pl.pallas_call(
    _enc_body,
    grid=(_G,),
    in_specs=[pl.BlockSpec((_R, 128), lambda i: (i, 0)),
              pl.BlockSpec((9, 64, H), lambda i: (0, 0, 0)),
              _row_spec],
    out_specs=_row_spec,
    out_shape=jax.ShapeDtypeStruct((N, H), jnp.float32),
)


def _pack_pairs(x):
    bits = lax.bitcast_convert_type(x.astype(jnp.bfloat16), jnp.uint16)
    bits = bits.astype(jnp.uint32)
    return (bits[..., 0::2] | (bits[..., 1::2] << 16)).astype(jnp.int32)


def kernel(rand_x, rand_edge, edge_index, node_feat, edge_attr, atom_emb,
           bond_emb, W1, b1, g1, be1, W2, b2, go, bo, eps):
    f32 = jnp.float32
    src = edge_index[0]
    dst = edge_index[1]
    cid = edge_attr[:, 0] * 64 + edge_attr[:, 1] * 8 + edge_attr[:, 2]
    pad = EP - E
    srcp = jnp.concatenate([src, jnp.zeros((pad,), jnp.int32)])
    dstp = jnp.concatenate([dst, jnp.full((pad,), N, jnp.int32)])
    cidp = jnp.concatenate([cid, jnp.zeros((pad,), jnp.int32)])
    randp = jnp.concatenate([_pack_pairs(rand_edge),
                             jnp.zeros((pad, RW), jnp.int32)], axis=0).T

    ii = jnp.arange(512)
    combo = (bond_emb[:, 0, ii // 64, :] + bond_emb[:, 1, (ii // 8) % 8, :]
             + bond_emb[:, 2, ii % 8, :])
    comboP = jnp.transpose(_pack_pairs(combo), (0, 2, 1))
    comboP = comboP.reshape(L, CW * 512)
    zrows = jnp.zeros((RPT, H), f32)

    nfp = jnp.pad(node_feat, ((0, 0), (0, 128 - 9)))
    rxp = jnp.pad(rand_x, ((0, 0), (SH, 0)))
    embp = jnp.pad(atom_emb, ((0, 0), (0, 0), (0, RV)))

    h = _encoder(nfp, embp, rxp)
    for l in range(L):
        sc_out = _msg_kernel(h, srcp, dstp, cidp, randp, comboP[l], zrows)
        mlp = _mlp_relu if l < L - 1 else _mlp_last
        h = mlp(h, sc_out[0], sc_out[1], W1[l], b1[l][None], g1[l][None],
                be1[l][None], W2[l], b2[l][None], go[l][None], bo[l][None],
                eps[l].reshape(1, 1))
    return h


# speedup vs baseline: 2.7475x; 1.4153x over previous
import functools

import jax
import jax.numpy as jnp
from jax import lax
from jax.experimental import pallas as pl
from jax.experimental.pallas import tpu as pltpu
from jax.experimental.pallas import tpu_sc as plsc

N = 10000
E = 320000
H = 128
RV = 10
SH = H - RV
L = 3

NC = 2
NS = 16
NW = NC * NS
CH = 128
NCHUNK = 79
EW = NCHUNK * CH
EP = NW * EW
NP2 = 10112
RPT = NP2 // NS
CW = SH // 2
RW = RV // 2

_sc_mesh = plsc.VectorSubcoreMesh(core_axis_name="c", subcore_axis_name="s")


def _unpack2(w):
    lo = plsc.bitcast(jnp.left_shift(w, 16), jnp.float32)
    hi = plsc.bitcast(jnp.bitwise_and(w, -65536), jnp.float32)
    return lo, hi


def _msg_body(h, srcp, dstp, cidp, randp, comboP, zrows, out,
              shared, combo_v, ibuf, hbuf, rbuf, sem):
    c = lax.axis_index("c")
    s = lax.axis_index("s")
    pltpu.sync_copy(comboP, combo_v)
    pltpu.sync_copy(zrows, shared.at[pl.ds(s * RPT, RPT)])
    plsc.subcore_barrier()

    base0 = (c * NS + s) * EW

    def chunk_body(i, carry):
        base = pl.multiple_of(base0 + i * CH, CH)
        pltpu.sync_copy(srcp.at[pl.ds(base, CH)], ibuf.at[0])
        pltpu.sync_copy(dstp.at[pl.ds(base, CH)], ibuf.at[1])
        pltpu.sync_copy(cidp.at[pl.ds(base, CH)], ibuf.at[2])
        pltpu.sync_copy(randp.at[:, pl.ds(base, CH)], rbuf)
        pltpu.async_copy(h.at[ibuf.at[0]], hbuf, sem).wait()

        for g in range(CH // 16):
            gbase = g * 16
            cvec = ibuf[2, pl.ds(gbase, 16)]
            rows = lax.iota(jnp.int32, 16) + gbase
            zero16 = jnp.zeros((16,), jnp.int32)

            def pair(d0, c0, c1, rows=rows, zero16=zero16):
                dsp0 = zero16 + d0
                dsp1 = dsp0 + 1
                m0 = jnp.maximum(plsc.load_gather(hbuf, [rows, dsp0]) + c0, 0.0)
                m1 = jnp.maximum(plsc.load_gather(hbuf, [rows, dsp1]) + c1, 0.0)
                plsc.store_scatter(hbuf, [rows, dsp0], m0)
                plsc.store_scatter(hbuf, [rows, dsp1], m1)

            def combo_word(w, cvec=cvec, pair=pair):
                cw = plsc.load_gather(combo_v, [w * 512 + cvec])
                clo, chi = _unpack2(cw)
                pair(2 * w, clo, chi)

            @plsc.parallel_loop(0, 56, unroll=8)
            def _(w, combo_word=combo_word):
                combo_word(w)

            for w in range(56, CW):
                combo_word(w)
            for j in range(RW):
                rw = plsc.load_gather(rbuf, [jnp.full((16,), j, jnp.int32), rows])
                rlo, rhi = _unpack2(rw)
                pair(SH + 2 * j, rlo, rhi)
        pltpu.sync_copy(hbuf, shared.at[ibuf.at[1]], add=True)
        return carry

    lax.fori_loop(0, NCHUNK, chunk_body, 0)
    plsc.subcore_barrier()

    nfull = RPT // CH
    for k in range(nfull):
        rb = s * RPT + k * CH
        pltpu.sync_copy(shared.at[pl.ds(rb, CH)], hbuf)
        pltpu.sync_copy(hbuf, out.at[c, pl.ds(rb, CH)])
    rem = RPT - nfull * CH
    if rem:
        rb = s * RPT + nfull * CH
        pltpu.sync_copy(shared.at[pl.ds(rb, rem)], hbuf.at[pl.ds(0, rem)])
        pltpu.sync_copy(hbuf.at[pl.ds(0, rem)], out.at[c, pl.ds(rb, rem)])


_msg_kernel = functools.partial(
    pl.kernel,
    mesh=_sc_mesh,
    compiler_params=pltpu.CompilerParams(needs_layout_passes=False),
    out_type=jax.ShapeDtypeStruct((NC, NP2, H), jnp.float32),
    scratch_types=[
        pltpu.VMEM_SHARED((NP2, H), jnp.float32),
        pltpu.VMEM((CW * 512,), jnp.int32),
        pltpu.VMEM((3, CH), jnp.int32),
        pltpu.VMEM((CH, H), jnp.float32),
        pltpu.VMEM((RW, CH), jnp.int32),
        pltpu.SemaphoreType.DMA,
    ],
)(_msg_body)


def _enc_body(nf_ref, emb_ref, rx_ref, out_ref):
    acc = rx_ref[...]
    nf = nf_ref[...]
    for i in range(9):
        col = nf[:, i][:, None]
        oh = (col == lax.broadcasted_iota(jnp.int32, (1, 64), 1)).astype(jnp.float32)
        acc = acc + jnp.dot(oh, emb_ref[i], preferred_element_type=jnp.float32)
    out_ref[...] = acc


def _mlp_body(h_ref, a0_ref, a1_ref, w1_ref, b1_ref, g1_ref, be1_ref,
              w2_ref, b2_ref, go_ref, bo_ref, eps_ref, out_ref, *, relu_out):
    x = h_ref[...]
    t = (1.0 + eps_ref[0, 0]) * x + a0_ref[...] + a1_ref[...]
    u = jnp.dot(t, w1_ref[...], preferred_element_type=jnp.float32) + b1_ref[...]
    u = jnp.maximum(g1_ref[...] * u + be1_ref[...], 0.0)
    v = jnp.dot(u, w2_ref[...], preferred_element_type=jnp.float32) + b2_ref[...]
    v = go_ref[...] * v + bo_ref[...]
    if relu_out:
        v = jnp.maximum(v, 0.0)
    out_ref[...] = v


_R = 400
_G = N // _R

_row_spec = pl.BlockSpec((_R, H), lambda i: (i, 0))
_vec_spec = pl.BlockSpec((1, H), lambda i: (0, 0))
_mat_spec = pl.BlockSpec((H, H), lambda i: (0, 0))


def _make_mlp(relu_out):
    return pl.pallas_call(
        functools.partial(_mlp_body, relu_out=relu_out),
        grid=(_G,),
        in_specs=[_row_spec, _row_spec, _row_spec, _mat_spec, _vec_spec,
                  _vec_spec, _vec_spec, _mat_spec, _vec_spec, _vec_spec,
                  _vec_spec, pl.BlockSpec((1, 1), lambda i: (0, 0))],
        out_specs=_row_spec,
        out_shape=jax.ShapeDtypeStruct((N, H), jnp.float32),
    )


_mlp_relu = _make_mlp(True)
_mlp_last = _make_mlp(False)

_encoder = pl.pallas_call(
    _enc_body,
    grid=(_G,),
    in_specs=[pl.BlockSpec((_R, 128), lambda i: (i, 0)),
              pl.BlockSpec((9, 64, H), lambda i: (0, 0, 0)),
              _row_spec],
    out_specs=_row_spec,
    out_shape=jax.ShapeDtypeStruct((N, H), jnp.float32),
)


def _pack_pairs(x):
    bits = lax.bitcast_convert_type(x.astype(jnp.bfloat16), jnp.uint16)
    bits = bits.astype(jnp.uint32)
    return (bits[..., 0::2] | (bits[..., 1::2] << 16)).astype(jnp.int32)


def kernel(rand_x, rand_edge, edge_index, node_feat, edge_attr, atom_emb,
           bond_emb, W1, b1, g1, be1, W2, b2, go, bo, eps):
    f32 = jnp.float32
    src = edge_index[0]
    dst = edge_index[1]
    cid = edge_attr[:, 0] * 64 + edge_attr[:, 1] * 8 + edge_attr[:, 2]
    pad = EP - E
    srcp = jnp.concatenate([src, jnp.zeros((pad,), jnp.int32)])
    dstp = jnp.concatenate([dst, jnp.full((pad,), N, jnp.int32)])
    cidp = jnp.concatenate([cid, jnp.zeros((pad,), jnp.int32)])
    randp = jnp.concatenate([_pack_pairs(rand_edge),
                             jnp.zeros((pad, RW), jnp.int32)], axis=0).T

    ii = jnp.arange(512)
    combo = (bond_emb[:, 0, ii // 64, :] + bond_emb[:, 1, (ii // 8) % 8, :]
             + bond_emb[:, 2, ii % 8, :])
    comboP = jnp.transpose(_pack_pairs(combo), (0, 2, 1))
    comboP = comboP.reshape(L, CW * 512)
    zrows = jnp.zeros((RPT, H), f32)

    nfp = jnp.pad(node_feat, ((0, 0), (0, 128 - 9)))
    rxp = jnp.pad(rand_x, ((0, 0), (SH, 0)))
    embp = jnp.pad(atom_emb, ((0, 0), (0, 0), (0, RV)))

    h = _encoder(nfp, embp, rxp)
    for l in range(L):
        sc_out = _msg_kernel(h, srcp, dstp, cidp, randp, comboP[l], zrows)
        mlp = _mlp_relu if l < L - 1 else _mlp_last
        h = mlp(h, sc_out[0], sc_out[1], W1[l], b1[l][None], g1[l][None],
                be1[l][None], W2[l], b2[l][None], go[l][None], bo[l][None],
                eps[l].reshape(1, 1))
    return h


# speedup vs baseline: 3.5192x; 1.2809x over previous
import functools

import jax
import jax.numpy as jnp
from jax import lax
from jax.experimental import pallas as pl
from jax.experimental.pallas import tpu as pltpu
from jax.experimental.pallas import tpu_sc as plsc

N = 10000
E = 320000
H = 128
RV = 10
SH = H - RV
L = 3

NC = 2
NS = 16
NW = NC * NS
CH = 48
NG = CH // 16
NCHUNK = 210
EW = NCHUNK * CH
EP = NW * EW
NCHT = NW * NCHUNK
NP2 = 10112
RPT = NP2 // NS
CW = SH // 2
RW = RV // 2
EREC = 8 * CH

_sc_mesh = plsc.VectorSubcoreMesh(core_axis_name="c", subcore_axis_name="s")


def _unpack2(w):
    lo = plsc.bitcast(jnp.left_shift(w, 16), jnp.float32)
    hi = plsc.bitcast(jnp.bitwise_and(w, -65536), jnp.float32)
    return lo, hi


def _compute_chunk(eb, hb, combo_v):
    for g in range(NG):
        gbase = g * 16
        cvec = eb[pl.ds(2 * CH + gbase, 16)]
        rows = lax.iota(jnp.int32, 16) + gbase
        zero16 = jnp.zeros((16,), jnp.int32)

        def pair(d0, c0, c1, rows=rows, zero16=zero16):
            dsp0 = zero16 + d0
            dsp1 = dsp0 + 1
            m0 = jnp.maximum(plsc.load_gather(hb, [rows, dsp0]) + c0, 0.0)
            m1 = jnp.maximum(plsc.load_gather(hb, [rows, dsp1]) + c1, 0.0)
            plsc.store_scatter(hb, [rows, dsp0], m0)
            plsc.store_scatter(hb, [rows, dsp1], m1)

        def combo_word(w, cvec=cvec, pair=pair):
            cw = plsc.load_gather(combo_v, [w * 512 + cvec])
            clo, chi = _unpack2(cw)
            pair(2 * w, clo, chi)

        @plsc.parallel_loop(0, 56, unroll=8)
        def _(w, combo_word=combo_word):
            combo_word(w)

        for w in range(56, CW):
            combo_word(w)
        for j in range(RW):
            rw = eb[pl.ds((3 + j) * CH + gbase, 16)]
            rlo, rhi = _unpack2(rw)
            pair(SH + 2 * j, rlo, rhi)


def _msg_body(h, edata, comboP, zrows, out,
              shared, combo_v, ebufA, ebufB, dbuf, hbufA, hbufB,
              gsemA, gsemB, lsemA, lsemB, sem_s):
    c = lax.axis_index("c")
    s = lax.axis_index("s")
    pltpu.sync_copy(comboP, combo_v)
    pltpu.sync_copy(zrows, shared.at[pl.ds(s * RPT, RPT)])
    plsc.subcore_barrier()

    ci0 = (c * NS + s) * NCHUNK
    ebufs = (ebufA, ebufB)
    hbufs = (hbufA, hbufB)
    gsems = (gsemA, gsemB)
    lsems = (lsemA, lsemB)

    def src_idx(eb):
        return eb.at[pl.ds(0, CH)]

    pltpu.sync_copy(edata.at[pl.ds(ci0 * EREC, EREC)], ebufA)
    pltpu.async_copy(edata.at[pl.ds((ci0 + 1) * EREC, EREC)], ebufB, lsemB)
    pltpu.async_copy(h.at[src_idx(ebufA)], hbufA, gsemA)

    def pair_body(p, carry):
        for b in range(2):
            eb, hb = ebufs[b], hbufs[b]
            ebn, hbn = ebufs[1 - b], hbufs[1 - b]
            j = p * 2 + b

            def wait_scatter(hbn=hbn, b=b):
                pltpu.make_async_copy(
                    hbn, shared.at[dbuf.at[1 - b]], sem_s).wait()
            if b == 0:
                @pl.when(p > 0)
                def _(wait_scatter=wait_scatter):
                    wait_scatter()
            else:
                wait_scatter()
            pltpu.make_async_copy(edata.at[pl.ds(0, EREC)], ebn,
                                  lsems[1 - b]).wait()
            pltpu.async_copy(h.at[src_idx(ebn)], hbn, gsems[1 - b])
            pltpu.make_async_copy(h.at[src_idx(eb)], hb, gsems[b]).wait()
            for k in range(NG):
                dbuf[b, pl.ds(k * 16, 16)] = eb[pl.ds(CH + k * 16, 16)]
            _compute_chunk(eb, hb, combo_v)
            ci2 = ci0 + jnp.minimum(j + 2, NCHUNK - 1)
            pltpu.async_copy(edata.at[pl.ds(ci2 * EREC, EREC)], eb, lsems[b])
            pltpu.async_copy(hb, shared.at[dbuf.at[b]], sem_s, add=True)
        return carry

    lax.fori_loop(0, NCHUNK // 2, pair_body, 0)
    pltpu.make_async_copy(hbufB, shared.at[dbuf.at[1]], sem_s).wait()
    pltpu.make_async_copy(h.at[src_idx(ebufA)], hbufA, gsemA).wait()
    pltpu.make_async_copy(edata.at[pl.ds(0, EREC)], ebufB, lsemB).wait()
    plsc.subcore_barrier()

    rb = s * RPT
    pltpu.sync_copy(shared.at[pl.ds(rb, RPT)], out.at[c, pl.ds(rb, RPT)])


_msg_kernel = functools.partial(
    pl.kernel,
    mesh=_sc_mesh,
    compiler_params=pltpu.CompilerParams(needs_layout_passes=False),
    out_type=jax.ShapeDtypeStruct((NC, NP2, H), jnp.float32),
    scratch_types=[
        pltpu.VMEM_SHARED((NP2, H), jnp.float32),
        pltpu.VMEM((CW * 512,), jnp.int32),
        pltpu.VMEM((EREC,), jnp.int32),
        pltpu.VMEM((EREC,), jnp.int32),
        pltpu.VMEM((2, CH), jnp.int32),
        pltpu.VMEM((CH, H), jnp.float32),
        pltpu.VMEM((CH, H), jnp.float32),
        pltpu.SemaphoreType.DMA,
        pltpu.SemaphoreType.DMA,
        pltpu.SemaphoreType.DMA,
        pltpu.SemaphoreType.DMA,
        pltpu.SemaphoreType.DMA,
    ],
)(_msg_body)


def _enc_body(nf_ref, emb_ref, rx_ref, out_ref):
    acc = rx_ref[...]
    nf = nf_ref[...]
    for i in range(9):
        col = nf[:, i][:, None]
        oh = (col == lax.broadcasted_iota(jnp.int32, (1, 64), 1)).astype(jnp.float32)
        acc = acc + jnp.dot(oh, emb_ref[i], preferred_element_type=jnp.float32)
    out_ref[...] = acc


def _mlp_body(h_ref, a0_ref, a1_ref, w1_ref, b1_ref, g1_ref, be1_ref,
              w2_ref, b2_ref, go_ref, bo_ref, eps_ref, out_ref, *, relu_out):
    x = h_ref[...]
    t = (1.0 + eps_ref[0, 0]) * x + a0_ref[...] + a1_ref[...]
    u = jnp.dot(t, w1_ref[...], preferred_element_type=jnp.float32) + b1_ref[...]
    u = jnp.maximum(g1_ref[...] * u + be1_ref[...], 0.0)
    v = jnp.dot(u, w2_ref[...], preferred_element_type=jnp.float32) + b2_ref[...]
    v = go_ref[...] * v + bo_ref[...]
    if relu_out:
        v = jnp.maximum(v, 0.0)
    out_ref[...] = v


_R = 400
_G = N // _R

_row_spec = pl.BlockSpec((_R, H), lambda i: (i, 0))
_vec_spec = pl.BlockSpec((1, H), lambda i: (0, 0))
_mat_spec = pl.BlockSpec((H, H), lambda i: (0, 0))


def _make_mlp(relu_out):
    return pl.pallas_call(
        functools.partial(_mlp_body, relu_out=relu_out),
        grid=(_G,),
        in_specs=[_row_spec, _row_spec, _row_spec, _mat_spec, _vec_spec,
                  _vec_spec, _vec_spec, _mat_spec, _vec_spec, _vec_spec,
                  _vec_spec, pl.BlockSpec((1, 1), lambda i: (0, 0))],
        out_specs=_row_spec,
        out_shape=jax.ShapeDtypeStruct((N, H), jnp.float32),
    )


_mlp_relu = _make_mlp(True)
_mlp_last = _make_mlp(False)

_encoder = pl.pallas_call(
    _enc_body,
    grid=(_G,),
    in_specs=[pl.BlockSpec((_R, 128), lambda i: (i, 0)),
              pl.BlockSpec((9, 64, H), lambda i: (0, 0, 0)),
              _row_spec],
    out_specs=_row_spec,
    out_shape=jax.ShapeDtypeStruct((N, H), jnp.float32),
)


def _pack_pairs(x):
    bits = lax.bitcast_convert_type(x.astype(jnp.bfloat16), jnp.uint16)
    bits = bits.astype(jnp.uint32)
    return (bits[..., 0::2] | (bits[..., 1::2] << 16)).astype(jnp.int32)


def kernel(rand_x, rand_edge, edge_index, node_feat, edge_attr, atom_emb,
           bond_emb, W1, b1, g1, be1, W2, b2, go, bo, eps):
    f32 = jnp.float32
    src = edge_index[0]
    dst = edge_index[1]
    cid = edge_attr[:, 0] * 64 + edge_attr[:, 1] * 8 + edge_attr[:, 2]
    pad = EP - E
    srcp = jnp.concatenate([src, jnp.zeros((pad,), jnp.int32)])
    dstp = jnp.concatenate([dst, jnp.full((pad,), N, jnp.int32)])
    cidp = jnp.concatenate([cid, jnp.zeros((pad,), jnp.int32)])
    randp = jnp.concatenate([_pack_pairs(rand_edge),
                             jnp.zeros((pad, RW), jnp.int32)], axis=0).T
    edata = jnp.concatenate([srcp[None], dstp[None], cidp[None], randp])
    edata = edata.reshape(8, NCHT, CH).transpose(1, 0, 2).reshape(-1)

    ii = jnp.arange(512)
    combo = (bond_emb[:, 0, ii // 64, :] + bond_emb[:, 1, (ii // 8) % 8, :]
             + bond_emb[:, 2, ii % 8, :])
    comboP = jnp.transpose(_pack_pairs(combo), (0, 2, 1))
    comboP = comboP.reshape(L, CW * 512)
    zrows = jnp.zeros((RPT, H), f32)

    nfp = jnp.pad(node_feat, ((0, 0), (0, 128 - 9)))
    rxp = jnp.pad(rand_x, ((0, 0), (SH, 0)))
    embp = jnp.pad(atom_emb, ((0, 0), (0, 0), (0, RV)))

    h = _encoder(nfp, embp, rxp)
    for l in range(L):
        sc_out = _msg_kernel(h, edata, comboP[l], zrows)
        mlp = _mlp_relu if l < L - 1 else _mlp_last
        h = mlp(h, sc_out[0], sc_out[1], W1[l], b1[l][None], g1[l][None],
                be1[l][None], W2[l], b2[l][None], go[l][None], bo[l][None],
                eps[l].reshape(1, 1))
    return h
